# baseline (speedup 1.0000x reference)
import functools

import jax
import jax.numpy as jnp
from jax import lax
from jax.experimental import pallas as pl
from jax.experimental.pallas import tpu as pltpu

N_TOKENS = 2048
D_IN = 1024
D_OUT = 1024
N_EXP = 8
BALANCE_W = 0.01
HALF_O = D_OUT // 2


def _moe_kernel(x_ref, gw_ref, ewa_ref, ewb_ref, eb_ref, y_ref, loss_ref,
                scores_ref, xw0_ref, xw1_ref):
    e = pl.program_id(0)

    @pl.when(e == 0)
    def _init():
        xf = x_ref[...]
        logits = lax.dot_general(
            xf, gw_ref[...], (((1,), (1,)), ((), ())),
            preferred_element_type=jnp.float32)
        idx = lax.broadcasted_iota(jnp.int32, logits.shape, 1)
        big = jnp.float32(3.4e38)
        m1 = jnp.max(logits, axis=1, keepdims=True)
        i1 = jnp.min(jnp.where(logits == m1, idx, N_EXP), axis=1,
                     keepdims=True)
        masked = jnp.where(idx == i1, -big, logits)
        m2 = jnp.max(masked, axis=1, keepdims=True)
        i2 = jnp.min(jnp.where(masked == m2, idx, N_EXP), axis=1,
                     keepdims=True)
        s2 = 1.0 / (1.0 + jnp.exp(m1 - m2))
        s1 = 1.0 - s2
        scores_ref[...] = jnp.where(
            idx == i1, s1, jnp.where(idx == i2, s2, 0.0))

    xf = x_ref[...]
    sc = scores_ref[...]

    @pl.when(lax.rem(e, 2) == 0)
    def _mm_even():
        xw0_ref[:, :HALF_O] = lax.dot_general(
            xf, ewa_ref[0], (((1,), (1,)), ((), ())),
            preferred_element_type=jnp.float32)
        xw0_ref[:, HALF_O:] = lax.dot_general(
            xf, ewb_ref[0], (((1,), (1,)), ((), ())),
            preferred_element_type=jnp.float32)

    @pl.when(lax.rem(e, 2) == 1)
    def _mm_odd():
        xw1_ref[:, :HALF_O] = lax.dot_general(
            xf, ewa_ref[0], (((1,), (1,)), ((), ())),
            preferred_element_type=jnp.float32)
        xw1_ref[:, HALF_O:] = lax.dot_general(
            xf, ewb_ref[0], (((1,), (1,)), ((), ())),
            preferred_element_type=jnp.float32)

    for k in range(N_EXP - 1):
        @pl.when(e == k + 1)
        def _apply(k=k):
            xw_prev = xw0_ref if k % 2 == 0 else xw1_ref
            s_col = sc[:, k:k + 1]
            if k == 0:
                y_ref[...] = s_col * xw_prev[...]
            else:
                y_ref[...] += s_col * xw_prev[...]

    @pl.when(e == N_EXP - 1)
    def _fini():
        y_ref[...] += sc[:, N_EXP - 1:] * xw1_ref[...]
        y_ref[...] += lax.dot_general(
            sc, eb_ref[...], (((1,), (0,)), ((), ())),
            preferred_element_type=jnp.float32)

        importance = jnp.sum(sc, axis=0)
        load = jnp.sum((sc > 0).astype(jnp.float32), axis=0)

        def cv_sq(v):
            mean = jnp.mean(v)
            var = jnp.sum((v - mean) ** 2) / (N_EXP - 1)
            return var / (mean * mean + 1e-10)

        loss = BALANCE_W * (cv_sq(importance) + cv_sq(load))
        loss_ref[...] = jnp.reshape(loss, (1, 1))


@functools.partial(jax.jit)
def _moe(xf, gate_W, expert_W, expert_b):
    y, loss = pl.pallas_call(
        _moe_kernel,
        grid=(N_EXP,),
        in_specs=[
            pl.BlockSpec((N_TOKENS, D_IN), lambda e: (0, 0)),
            pl.BlockSpec((N_EXP, D_IN), lambda e: (0, 0)),
            pl.BlockSpec((1, HALF_O, D_IN), lambda e: (e, 0, 0)),
            pl.BlockSpec((1, HALF_O, D_IN), lambda e: (e, 1, 0)),
            pl.BlockSpec((N_EXP, D_OUT), lambda e: (0, 0)),
        ],
        out_specs=[
            pl.BlockSpec((N_TOKENS, D_OUT), lambda e: (0, 0)),
            pl.BlockSpec((1, 1), lambda e: (0, 0)),
        ],
        out_shape=[
            jax.ShapeDtypeStruct((N_TOKENS, D_OUT), jnp.float32),
            jax.ShapeDtypeStruct((1, 1), jnp.float32),
        ],
        scratch_shapes=[
            pltpu.VMEM((N_TOKENS, N_EXP), jnp.float32),
            pltpu.VMEM((N_TOKENS, D_OUT), jnp.float32),
            pltpu.VMEM((N_TOKENS, D_OUT), jnp.float32),
        ],
    )(xf, gate_W, expert_W, expert_W, expert_b)
    return y, loss


def kernel(x, gate_W, expert_W, expert_b):
    orig_shape = x.shape[:-1]
    xf = x.reshape(-1, D_IN)
    y, loss = _moe(xf, gate_W, expert_W, expert_b)
    return y.reshape(orig_shape + (D_OUT,)), loss[0, 0]
